# baseline (speedup 1.0000x reference)
import jax
import jax.numpy as jnp
from jax import lax
from jax.experimental import pallas as pl
from jax.experimental.pallas import tpu as pltpu

_DIM = 512
_NUM_ITEM = 4096
_K = 16
_TR = 256


def _tile_body(x_ref, mp_ref, mpb_ref, out1_ref, out2_ref, att_s):
    j = pl.program_id(0)
    nt = pl.num_programs(0) - 1

    @pl.when(j < nt)
    def _produce():
        qc = x_ref[0]
        att = lax.dot_general(
            qc,
            mp_ref[...],
            (((0,), (1,)), ((), ())),
            preferred_element_type=jnp.float32,
        )
        att_s[pl.ds(j % 2, 1)] = att[None]

    @pl.when(j > 0)
    def _consume():
        att = att_s[(j - 1) % 2]
        m0 = jnp.max(att, axis=1, keepdims=True)
        denom = jnp.ones((_TR, 1), jnp.float32)
        m = m0
        for _ in range(_K - 1):
            m = jnp.max(
                jnp.where(att < m, att, -jnp.inf), axis=1, keepdims=True
            )
            denom = denom + jnp.exp(m - m0)
        t = m
        p = jnp.where(att >= t, jnp.exp(att - m0), 0.0)
        recip = 1.0 / denom
        out2_ref[...] = p * att * recip
        out1t = lax.dot_general(
            mpb_ref[...],
            p.astype(jnp.bfloat16),
            (((0,), (1,)), ((), ())),
            preferred_element_type=jnp.float32,
        )
        out1_ref[0] = out1t * jnp.reshape(recip, (1, _TR))


def kernel(input, mempool):
    B, C, H, W = input.shape
    x3 = input.reshape(B, C, H * W)
    rows = B * H * W
    ntiles = rows // _TR
    tpb = (H * W) // _TR

    def prod_map(i):
        ic = jnp.minimum(i, ntiles - 1)
        return (ic // tpb, 0, ic % tpb)

    def cons_map3(i):
        ic = jnp.maximum(i - 1, 0)
        return (ic // tpb, 0, ic % tpb)

    def cons_map2(i):
        return (jnp.maximum(i - 1, 0), 0)

    out1, out2 = pl.pallas_call(
        _tile_body,
        grid=(ntiles + 1,),
        in_specs=[
            pl.BlockSpec((1, C, _TR), prod_map),
            pl.BlockSpec((_NUM_ITEM, C), lambda i: (0, 0)),
            pl.BlockSpec((_NUM_ITEM, C), lambda i: (0, 0)),
        ],
        out_specs=[
            pl.BlockSpec((1, C, _TR), cons_map3),
            pl.BlockSpec((_TR, _NUM_ITEM), cons_map2),
        ],
        out_shape=[
            jax.ShapeDtypeStruct((B, C, H * W), jnp.float32),
            jax.ShapeDtypeStruct((rows, _NUM_ITEM), jnp.float32),
        ],
        scratch_shapes=[
            pltpu.VMEM((2, _TR, _NUM_ITEM), jnp.float32),
        ],
        compiler_params=pltpu.CompilerParams(
            dimension_semantics=("arbitrary",),
        ),
    )(x3, mempool, mempool.astype(jnp.bfloat16))
    return out1.reshape(B, C, H, W), out2
